# baseline (speedup 1.0000x reference)
import functools

import jax
import jax.numpy as jnp
from jax import lax
from jax.experimental import pallas as pl
from jax.experimental.pallas import tpu as pltpu
from jax.experimental.pallas import tpu_sc as plsc

_B, _OBS, _HID, _LAT, _V = 8192, 512, 1024, 64, 8192
_EPS = 1e-5
_BT_ENC = 256
_BT_DEC = 2048
_NW = 32
_BPW = _B // _NW
_KAUG = 72


def _enc_argmin_body(x_ref, w1_ref, b1_ref, g_ref, bln_ref, w2_ref, b2_ref,
                     embt_ref, lat_ref, idx_ref, aug_ref, w1b_ref, w2b_ref,
                     w1m_ref, ones_ref):
    @pl.when(pl.program_id(0) == 0)
    def _prep():
        w1b_ref[...] = w1_ref[...].astype(jnp.bfloat16)
        w2b_ref[...] = w2_ref[...].astype(jnp.bfloat16)
        w1m = jnp.sum(w1_ref[...], axis=1, keepdims=True) * (1.0 / _HID)
        w1m_ref[...] = jnp.broadcast_to(w1m, (_OBS, 128)).astype(jnp.bfloat16)
        ones_ref[...] = jnp.ones((_HID, 128), jnp.bfloat16)
        embt = embt_ref[...]
        aug_ref[0:_LAT, :] = (embt * -2.0).astype(jnp.bfloat16)
        aug_ref[_LAT:_LAT + 1, :] = jnp.ones((1, _V), jnp.bfloat16)
        esq = jnp.sum(embt * embt, axis=0, keepdims=True)
        aug_ref[_LAT + 1:_LAT + 2, :] = (esq + 1.0).astype(jnp.bfloat16)
        aug_ref[_LAT + 2:_KAUG, :] = jnp.zeros((_KAUG - _LAT - 2, _V),
                                               jnp.bfloat16)

    xb = x_ref[...].astype(jnp.bfloat16)
    h = jnp.dot(xb, w1b_ref[...], preferred_element_type=jnp.float32)
    h = h + b1_ref[...]
    mu = (jnp.dot(xb, w1m_ref[...], preferred_element_type=jnp.float32)[:, :1]
          + jnp.mean(b1_ref[...]))
    hb = h.astype(jnp.bfloat16)
    s2 = jnp.dot(hb * hb, ones_ref[...],
                 preferred_element_type=jnp.float32)[:, :1] * (1.0 / _HID)
    var = s2 - mu * mu
    h = (h - mu) / jnp.sqrt(var + _EPS) * g_ref[...] + bln_ref[...]
    h = jnp.maximum(h, 0.0)
    lat = jnp.dot(h.astype(jnp.bfloat16), w2b_ref[...],
                  preferred_element_type=jnp.float32)
    lat = lat + b2_ref[...]
    lat_ref[...] = lat
    lsq = jnp.sum(lat * lat, axis=1, keepdims=True)
    lat_aug = jnp.concatenate(
        [lat, lsq, jnp.ones((_BT_ENC, 1), jnp.float32),
         jnp.zeros((_BT_ENC, _KAUG - _LAT - 2), jnp.float32)], axis=1)
    d2p = jnp.dot(lat_aug.astype(jnp.bfloat16), aug_ref[...],
                  preferred_element_type=jnp.float32)
    ii = lax.broadcasted_iota(jnp.int32, d2p.shape, 1)
    bits = lax.bitcast_convert_type(d2p, jnp.int32)
    key = lax.bitcast_convert_type((bits & jnp.int32(-8192)) | ii,
                                   jnp.float32)
    kmin = jnp.min(key, axis=1)
    idx_ref[...] = lax.bitcast_convert_type(kmin, jnp.int32) & 8191


def _encode_and_argmin(x, emb_t, W1, b1, ln_g, ln_b, W2, b2):
    b = x.shape[0]
    nb = b // _BT_ENC
    return pl.pallas_call(
        _enc_argmin_body,
        grid=(nb,),
        in_specs=[
            pl.BlockSpec((_BT_ENC, _OBS), lambda i: (i, 0)),
            pl.BlockSpec((_OBS, _HID), lambda i: (0, 0)),
            pl.BlockSpec((1, _HID), lambda i: (0, 0)),
            pl.BlockSpec((1, _HID), lambda i: (0, 0)),
            pl.BlockSpec((1, _HID), lambda i: (0, 0)),
            pl.BlockSpec((_HID, _LAT), lambda i: (0, 0)),
            pl.BlockSpec((1, _LAT), lambda i: (0, 0)),
            pl.BlockSpec((_LAT, _V), lambda i: (0, 0)),
        ],
        out_specs=[
            pl.BlockSpec((_BT_ENC, _LAT), lambda i: (i, 0)),
            pl.BlockSpec((_BT_ENC,), lambda i: (i,)),
        ],
        out_shape=[
            jax.ShapeDtypeStruct((b, _LAT), jnp.float32),
            jax.ShapeDtypeStruct((b,), jnp.int32),
        ],
        scratch_shapes=[pltpu.VMEM((_KAUG, _V), jnp.bfloat16),
                        pltpu.VMEM((_OBS, _HID), jnp.bfloat16),
                        pltpu.VMEM((_HID, _LAT), jnp.bfloat16),
                        pltpu.VMEM((_OBS, 128), jnp.bfloat16),
                        pltpu.VMEM((_HID, 128), jnp.bfloat16)],
    )(x, W1, b1.reshape(1, _HID), ln_g.reshape(1, _HID), ln_b.reshape(1, _HID),
      W2, b2.reshape(1, _LAT), emb_t)


def _gather_rows_sc(emb_W, idx):
    b = idx.shape[0]
    bpw = b // _NW
    mesh = plsc.VectorSubcoreMesh(core_axis_name="c", subcore_axis_name="s")

    @functools.partial(
        pl.kernel, mesh=mesh,
        compiler_params=pltpu.CompilerParams(use_tc_tiling_on_sc=False),
        out_type=jax.ShapeDtypeStruct((b, _LAT), jnp.float32),
        scratch_types=[
            pltpu.VMEM((bpw,), jnp.int32),
            pltpu.VMEM((bpw, _LAT), jnp.float32),
            pltpu.VMEM_SHARED((_V, _LAT), jnp.float32),
            pltpu.SemaphoreType.DMA,
        ],
    )
    def k(table_hbm, idx_hbm, out_hbm, idx_v, rows_v, table_sp, sem):
        sid = lax.axis_index("s")
        wid = sid * 2 + lax.axis_index("c")
        base = wid * bpw

        @pl.when(sid == 0)
        def _stage():
            pltpu.sync_copy(table_hbm, table_sp)

        pltpu.sync_copy(idx_hbm.at[pl.ds(base, bpw)], idx_v)
        plsc.subcore_barrier()
        pltpu.async_copy(table_sp.at[idx_v], rows_v, sem).wait()
        pltpu.sync_copy(rows_v, out_hbm.at[pl.ds(base, bpw)])

    return k(emb_W, idx)


def _dec_loss_body(x_ref, lat_ref, q_ref, w3_ref, b3_ref, w4_ref, b4_ref,
                   out_ref, acc_ref, w3b_ref, w4b_ref):
    i = pl.program_id(0)

    @pl.when(i == 0)
    def _init():
        acc_ref[0] = 0.0
        acc_ref[1] = 0.0
        w3b_ref[...] = w3_ref[...].astype(jnp.bfloat16)
        w4b_ref[...] = w4_ref[...].astype(jnp.bfloat16)

    lat = lat_ref[...]
    qst = lat + (q_ref[...] - lat)
    d = jnp.dot(qst.astype(jnp.bfloat16), w3b_ref[...],
                preferred_element_type=jnp.float32)
    d = jnp.maximum(d + b3_ref[...], 0.0)
    rec = jnp.dot(d.astype(jnp.bfloat16), w4b_ref[...],
                  preferred_element_type=jnp.float32)
    rec = rec + b4_ref[...]
    dx = rec - x_ref[...]
    dq = qst - lat
    acc_ref[0] += jnp.sum(dx * dx)
    acc_ref[1] += jnp.sum(dq * dq)

    @pl.when(i == pl.num_programs(0) - 1)
    def _fin():
        out_ref[0, 0] = (0.5 * acc_ref[0] / (_B * _OBS)
                         + 1.5 * acc_ref[1] / (_B * _LAT))


def _decode_and_loss(x, latent, quantised, W3, b3, W4, b4):
    nb = x.shape[0] // _BT_DEC
    out = pl.pallas_call(
        _dec_loss_body,
        grid=(nb,),
        in_specs=[
            pl.BlockSpec((_BT_DEC, _OBS), lambda i: (i, 0)),
            pl.BlockSpec((_BT_DEC, _LAT), lambda i: (i, 0)),
            pl.BlockSpec((_BT_DEC, _LAT), lambda i: (i, 0)),
            pl.BlockSpec((_LAT, _HID), lambda i: (0, 0)),
            pl.BlockSpec((1, _HID), lambda i: (0, 0)),
            pl.BlockSpec((_HID, _OBS), lambda i: (0, 0)),
            pl.BlockSpec((1, _OBS), lambda i: (0, 0)),
        ],
        out_specs=pl.BlockSpec(memory_space=pltpu.SMEM),
        out_shape=jax.ShapeDtypeStruct((1, 1), jnp.float32),
        scratch_shapes=[pltpu.SMEM((2,), jnp.float32),
                        pltpu.VMEM((_LAT, _HID), jnp.bfloat16),
                        pltpu.VMEM((_HID, _OBS), jnp.bfloat16)],
    )(x, latent, quantised, W3, b3.reshape(1, _HID), W4, b4.reshape(1, _OBS))
    return out[0, 0]


def kernel(x, emb_W, W1, b1, ln_g, ln_b, W2, b2, W3, b3, W4, b4):
    latent, idx = _encode_and_argmin(x, emb_W.T, W1, b1, ln_g, ln_b, W2, b2)
    quantised = _gather_rows_sc(emb_W, idx)
    return _decode_and_loss(x, latent, quantised, W3, b3, W4, b4)


# speedup vs baseline: 1.0180x; 1.0180x over previous
import functools

import jax
import jax.numpy as jnp
from jax import lax
from jax.experimental import pallas as pl
from jax.experimental.pallas import tpu as pltpu
from jax.experimental.pallas import tpu_sc as plsc

_B, _OBS, _HID, _LAT, _V = 8192, 512, 1024, 64, 8192
_EPS = 1e-5
_BT_ENC = 256
_BT_DEC = 2048
_NW = 32
_BPW = _B // _NW
_KAUG = 72


def _enc_argmin_body(x_ref, w1_ref, b1_ref, g_ref, bln_ref, w2_ref, b2_ref,
                     embt_ref, lat_ref, idx_ref, aug_ref, w1b_ref, w2b_ref):
    @pl.when(pl.program_id(0) == 0)
    def _prep():
        w1b_ref[...] = w1_ref[...].astype(jnp.bfloat16)
        w2b_ref[...] = w2_ref[...].astype(jnp.bfloat16)
        embt = embt_ref[...]
        aug_ref[0:_LAT, :] = (embt * -2.0).astype(jnp.bfloat16)
        aug_ref[_LAT:_LAT + 1, :] = jnp.ones((1, _V), jnp.bfloat16)
        esq = jnp.sum(embt * embt, axis=0, keepdims=True)
        aug_ref[_LAT + 1:_LAT + 2, :] = (esq + 1.0).astype(jnp.bfloat16)
        aug_ref[_LAT + 2:_KAUG, :] = jnp.zeros((_KAUG - _LAT - 2, _V),
                                               jnp.bfloat16)

    h = jnp.dot(x_ref[...].astype(jnp.bfloat16), w1b_ref[...],
                preferred_element_type=jnp.float32)
    h = h + b1_ref[...]
    mu = jnp.mean(h, axis=1, keepdims=True)
    var = jnp.mean((h - mu) ** 2, axis=1, keepdims=True)
    h = (h - mu) / jnp.sqrt(var + _EPS) * g_ref[...] + bln_ref[...]
    h = jnp.maximum(h, 0.0)
    lat = jnp.dot(h.astype(jnp.bfloat16), w2b_ref[...],
                  preferred_element_type=jnp.float32)
    lat = lat + b2_ref[...]
    lat_ref[...] = lat
    lsq = jnp.sum(lat * lat, axis=1, keepdims=True)
    lat_aug = jnp.concatenate(
        [lat, lsq, jnp.ones((_BT_ENC, 1), jnp.float32),
         jnp.zeros((_BT_ENC, _KAUG - _LAT - 2), jnp.float32)], axis=1)
    d2p = jnp.dot(lat_aug.astype(jnp.bfloat16), aug_ref[...],
                  preferred_element_type=jnp.float32)
    ii = lax.broadcasted_iota(jnp.int32, d2p.shape, 1)
    bits = lax.bitcast_convert_type(d2p, jnp.int32)
    key = lax.bitcast_convert_type((bits & jnp.int32(-8192)) | ii,
                                   jnp.float32)
    kmin = jnp.min(key, axis=1)
    idx_ref[...] = lax.bitcast_convert_type(kmin, jnp.int32) & 8191


def _encode_and_argmin(x, emb_t, W1, b1, ln_g, ln_b, W2, b2):
    b = x.shape[0]
    nb = b // _BT_ENC
    return pl.pallas_call(
        _enc_argmin_body,
        grid=(nb,),
        in_specs=[
            pl.BlockSpec((_BT_ENC, _OBS), lambda i: (i, 0)),
            pl.BlockSpec((_OBS, _HID), lambda i: (0, 0)),
            pl.BlockSpec((1, _HID), lambda i: (0, 0)),
            pl.BlockSpec((1, _HID), lambda i: (0, 0)),
            pl.BlockSpec((1, _HID), lambda i: (0, 0)),
            pl.BlockSpec((_HID, _LAT), lambda i: (0, 0)),
            pl.BlockSpec((1, _LAT), lambda i: (0, 0)),
            pl.BlockSpec((_LAT, _V), lambda i: (0, 0)),
        ],
        out_specs=[
            pl.BlockSpec((_BT_ENC, _LAT), lambda i: (i, 0)),
            pl.BlockSpec((_BT_ENC,), lambda i: (i,)),
        ],
        out_shape=[
            jax.ShapeDtypeStruct((b, _LAT), jnp.float32),
            jax.ShapeDtypeStruct((b,), jnp.int32),
        ],
        scratch_shapes=[pltpu.VMEM((_KAUG, _V), jnp.bfloat16),
                        pltpu.VMEM((_OBS, _HID), jnp.bfloat16),
                        pltpu.VMEM((_HID, _LAT), jnp.bfloat16)],
    )(x, W1, b1.reshape(1, _HID), ln_g.reshape(1, _HID), ln_b.reshape(1, _HID),
      W2, b2.reshape(1, _LAT), emb_t)


def _gather_rows_sc(emb_W, idx):
    b = idx.shape[0]
    bpw = b // _NW
    mesh = plsc.VectorSubcoreMesh(core_axis_name="c", subcore_axis_name="s")

    @functools.partial(
        pl.kernel, mesh=mesh,
        compiler_params=pltpu.CompilerParams(use_tc_tiling_on_sc=False),
        out_type=jax.ShapeDtypeStruct((b, _LAT), jnp.float32),
        scratch_types=[
            pltpu.VMEM((bpw,), jnp.int32),
            pltpu.VMEM((bpw, _LAT), jnp.float32),
            pltpu.VMEM_SHARED((_V, _LAT), jnp.float32),
            pltpu.SemaphoreType.DMA,
        ],
    )
    def k(table_hbm, idx_hbm, out_hbm, idx_v, rows_v, table_sp, sem):
        sid = lax.axis_index("s")
        wid = sid * 2 + lax.axis_index("c")
        base = wid * bpw

        @pl.when(sid == 0)
        def _stage():
            pltpu.sync_copy(table_hbm, table_sp)

        pltpu.sync_copy(idx_hbm.at[pl.ds(base, bpw)], idx_v)
        plsc.subcore_barrier()
        pltpu.async_copy(table_sp.at[idx_v], rows_v, sem).wait()
        pltpu.sync_copy(rows_v, out_hbm.at[pl.ds(base, bpw)])

    return k(emb_W, idx)


def _dec_loss_body(x_ref, lat_ref, q_ref, w3_ref, b3_ref, w4_ref, b4_ref,
                   out_ref, acc_ref, w3b_ref, w4b_ref):
    i = pl.program_id(0)

    @pl.when(i == 0)
    def _init():
        acc_ref[0] = 0.0
        acc_ref[1] = 0.0
        w3b_ref[...] = w3_ref[...].astype(jnp.bfloat16)
        w4b_ref[...] = w4_ref[...].astype(jnp.bfloat16)

    lat = lat_ref[...]
    qst = lat + (q_ref[...] - lat)
    d = jnp.dot(qst.astype(jnp.bfloat16), w3b_ref[...],
                preferred_element_type=jnp.float32)
    d = jnp.maximum(d + b3_ref[...], 0.0)
    rec = jnp.dot(d.astype(jnp.bfloat16), w4b_ref[...],
                  preferred_element_type=jnp.float32)
    rec = rec + b4_ref[...]
    dx = rec - x_ref[...]
    dq = qst - lat
    acc_ref[0] += jnp.sum(dx * dx)
    acc_ref[1] += jnp.sum(dq * dq)

    @pl.when(i == pl.num_programs(0) - 1)
    def _fin():
        out_ref[0, 0] = (0.5 * acc_ref[0] / (_B * _OBS)
                         + 1.5 * acc_ref[1] / (_B * _LAT))


def _decode_and_loss(x, latent, quantised, W3, b3, W4, b4):
    nb = x.shape[0] // _BT_DEC
    out = pl.pallas_call(
        _dec_loss_body,
        grid=(nb,),
        in_specs=[
            pl.BlockSpec((_BT_DEC, _OBS), lambda i: (i, 0)),
            pl.BlockSpec((_BT_DEC, _LAT), lambda i: (i, 0)),
            pl.BlockSpec((_BT_DEC, _LAT), lambda i: (i, 0)),
            pl.BlockSpec((_LAT, _HID), lambda i: (0, 0)),
            pl.BlockSpec((1, _HID), lambda i: (0, 0)),
            pl.BlockSpec((_HID, _OBS), lambda i: (0, 0)),
            pl.BlockSpec((1, _OBS), lambda i: (0, 0)),
        ],
        out_specs=pl.BlockSpec(memory_space=pltpu.SMEM),
        out_shape=jax.ShapeDtypeStruct((1, 1), jnp.float32),
        scratch_shapes=[pltpu.SMEM((2,), jnp.float32),
                        pltpu.VMEM((_LAT, _HID), jnp.bfloat16),
                        pltpu.VMEM((_HID, _OBS), jnp.bfloat16)],
    )(x, latent, quantised, W3, b3.reshape(1, _HID), W4, b4.reshape(1, _OBS))
    return out[0, 0]


def kernel(x, emb_W, W1, b1, ln_g, ln_b, W2, b2, W3, b3, W4, b4):
    latent, idx = _encode_and_argmin(x, emb_W.T, W1, b1, ln_g, ln_b, W2, b2)
    quantised = _gather_rows_sc(emb_W, idx)
    return _decode_and_loss(x, latent, quantised, W3, b3, W4, b4)
